# baseline (speedup 1.0000x reference)
import jax
import jax.numpy as jnp
from jax.experimental import pallas as pl
from jax.experimental.pallas import tpu as pltpu

H = 32
W = 32
D = 256
B = 8


def _body(col_ref, row_ref, out_ref, scratch, sems):
    col = col_ref[...]
    for h in range(H):
        scratch[h, :, :D] = col
        scratch[h, :, D:] = jnp.broadcast_to(row_ref[h, :][None, :], (W, D))
    copies = [
        pltpu.make_async_copy(
            scratch.at[pl.ds(16 * k, 16)],
            out_ref.at[b, pl.ds(16 * k, 16)],
            sems.at[2 * b + k],
        )
        for b in range(B)
        for k in range(2)
    ]
    for c in copies:
        c.start()
    for c in copies:
        c.wait()


def kernel(x, row_embed, col_embed):
    b = x.shape[0]
    out = pl.pallas_call(
        _body,
        grid=(1,),
        in_specs=[
            pl.BlockSpec((W, D), lambda i: (0, 0)),
            pl.BlockSpec((H, D), lambda i: (0, 0)),
        ],
        out_specs=pl.BlockSpec(memory_space=pl.ANY),
        out_shape=jax.ShapeDtypeStruct((b, H, W, 2 * D), jnp.float32),
        scratch_shapes=[
            pltpu.VMEM((H, W, 2 * D), jnp.float32),
            pltpu.SemaphoreType.DMA((2 * B,)),
        ],
    )(col_embed, row_embed)
    return jnp.transpose(out, (0, 3, 1, 2))


# speedup vs baseline: 1.0111x; 1.0111x over previous
import jax
import jax.numpy as jnp
from jax.experimental import pallas as pl
from jax.experimental.pallas import tpu as pltpu

H = 32
W = 32
D = 256
B = 8


def _body(col_ref, row_ref, out_ref, scratch, sems):
    col = col_ref[...]
    for h in range(H):
        scratch[h, :, :D] = col
        scratch[h, :, D:] = jnp.broadcast_to(row_ref[h, :][None, :], (W, D))
    copies = [
        pltpu.make_async_copy(scratch, out_ref.at[b], sems.at[b])
        for b in range(B)
    ]
    for c in copies:
        c.start()
    for c in copies:
        c.wait()


def kernel(x, row_embed, col_embed):
    b = x.shape[0]
    out = pl.pallas_call(
        _body,
        grid=(1,),
        in_specs=[
            pl.BlockSpec((W, D), lambda i: (0, 0)),
            pl.BlockSpec((H, D), lambda i: (0, 0)),
        ],
        out_specs=pl.BlockSpec(memory_space=pl.ANY),
        out_shape=jax.ShapeDtypeStruct((b, H, W, 2 * D), jnp.float32),
        scratch_shapes=[
            pltpu.VMEM((H, W, 2 * D), jnp.float32),
            pltpu.SemaphoreType.DMA((B,)),
        ],
    )(col_embed, row_embed)
    return jnp.transpose(out, (0, 3, 1, 2))
